# baseline (speedup 1.0000x reference)
import functools

import jax
import jax.numpy as jnp
from jax import lax
from jax.experimental import pallas as pl
from jax.experimental.pallas import tpu as pltpu
from jax.experimental.pallas import tpu_sc as plsc

B = 16
C = 3
N = 8192
M = 1024
L = 16
NCH = N // L


def _fps_body(x_hbm, y_hbm, xs_v, ys_v, zs_v, dist_v, idx_v,
              ox_v, oy_v, oz_v):
    cid = lax.axis_index("c")
    sid = lax.axis_index("s")
    wid = sid * 2 + cid

    @pl.when(wid < B)
    def _():
        row = wid * C
        pltpu.sync_copy(x_hbm.at[row + 0], xs_v)
        pltpu.sync_copy(x_hbm.at[row + 1], ys_v)
        pltpu.sync_copy(x_hbm.at[row + 2], zs_v)

        inf16 = jnp.full((L,), jnp.inf, jnp.float32)

        def init_j(j, carry):
            dist_v[pl.ds(j * L, L)] = inf16
            return carry

        lax.fori_loop(0, NCH, init_j, 0)
        idx_v[pl.ds(0, L)] = jnp.zeros((L,), jnp.int32)

        lane = lax.iota(jnp.int32, L)

        U = 4

        def step(t, k):
            lx = plsc.load_gather(xs_v, [k])
            ly = plsc.load_gather(ys_v, [k])
            lz = plsc.load_gather(zs_v, [k])

            def chunk(g, carry):
                bests, bidxs = carry
                nb, ni = [], []
                for u in range(U):
                    s = (g * U + u) * L
                    dx = xs_v[pl.ds(s, L)] - lx
                    dy = ys_v[pl.ds(s, L)] - ly
                    dz = zs_v[pl.ds(s, L)] - lz
                    d = dx * dx + dy * dy + dz * dz
                    nd = jnp.minimum(dist_v[pl.ds(s, L)], d)
                    dist_v[pl.ds(s, L)] = nd
                    m = nd > bests[u]
                    nb.append(jnp.where(m, nd, bests[u]))
                    ni.append(jnp.where(m, s + lane, bidxs[u]))
                return tuple(nb), tuple(ni)

            best0 = tuple(jnp.full((L,), -jnp.inf, jnp.float32)
                          for _ in range(U))
            bidx0 = tuple(jnp.zeros((L,), jnp.int32) for _ in range(U))
            bests, bidxs = lax.fori_loop(0, NCH // U, chunk, (best0, bidx0))

            def merge(a, b):
                (b1, i1), (b2, i2) = a, b
                gt = b2 > b1
                eq = b2 == b1
                return (jnp.where(gt, b2, b1),
                        jnp.where(gt, i2,
                                  jnp.where(eq, jnp.minimum(i1, i2), i1)))

            acc = list(zip(bests, bidxs))
            while len(acc) > 1:
                acc = [merge(acc[i], acc[i + 1])
                       for i in range(0, len(acc), 2)]
            best, bidx = acc[0]

            maxv = jnp.max(best)
            cand = jnp.where(best == maxv, bidx, jnp.int32(2**30))
            knext = jnp.full((L,), jnp.min(cand), jnp.int32)
            plsc.store_scatter(idx_v, [jnp.full((L,), t, jnp.int32)],
                               knext, mask=lane == 0)
            return knext

        lax.fori_loop(1, M, step, jnp.zeros((L,), jnp.int32))

        def gout(j, carry):
            s = j * L
            iv = idx_v[pl.ds(s, L)]
            ox_v[pl.ds(s, L)] = plsc.load_gather(xs_v, [iv])
            oy_v[pl.ds(s, L)] = plsc.load_gather(ys_v, [iv])
            oz_v[pl.ds(s, L)] = plsc.load_gather(zs_v, [iv])
            return carry

        lax.fori_loop(0, M // L, gout, 0)

        pltpu.sync_copy(ox_v, y_hbm.at[row + 0])
        pltpu.sync_copy(oy_v, y_hbm.at[row + 1])
        pltpu.sync_copy(oz_v, y_hbm.at[row + 2])


@jax.jit
def _fps_sc(xr):
    mesh = plsc.VectorSubcoreMesh(core_axis_name="c", subcore_axis_name="s")
    f = functools.partial(
        pl.kernel,
        mesh=mesh,
        compiler_params=pltpu.CompilerParams(needs_layout_passes=False),
        out_type=jax.ShapeDtypeStruct((B * C, M), jnp.float32),
        scratch_types=[
            pltpu.VMEM((N,), jnp.float32),
            pltpu.VMEM((N,), jnp.float32),
            pltpu.VMEM((N,), jnp.float32),
            pltpu.VMEM((N,), jnp.float32),
            pltpu.VMEM((M,), jnp.int32),
            pltpu.VMEM((M,), jnp.float32),
            pltpu.VMEM((M,), jnp.float32),
            pltpu.VMEM((M,), jnp.float32),
        ],
    )(_fps_body)
    return f(xr)


def kernel(x):
    xr = x.reshape(B * C, N)
    yr = _fps_sc(xr)
    return yr.reshape(B, C, M)


# speedup vs baseline: 2.7031x; 2.7031x over previous
import functools

import jax
import jax.numpy as jnp
from jax import lax
from jax.experimental import pallas as pl
from jax.experimental.pallas import tpu as pltpu
from jax.experimental.pallas import tpu_sc as plsc

B = 16
C = 3
N = 8192
M = 1024
L = 16
NCH = N // L


def _fps_body(x_hbm, y_hbm, xs_v, ys_v, zs_v, dist_v, idx_v,
              ox_v, oy_v, oz_v):
    cid = lax.axis_index("c")
    sid = lax.axis_index("s")
    wid = sid * 2 + cid

    @pl.when(wid < B)
    def _():
        row = wid * C
        pltpu.sync_copy(x_hbm.at[row + 0], xs_v)
        pltpu.sync_copy(x_hbm.at[row + 1], ys_v)
        pltpu.sync_copy(x_hbm.at[row + 2], zs_v)

        inf16 = jnp.full((L,), jnp.inf, jnp.float32)

        def init_j(j, carry):
            dist_v[pl.ds(j * L, L)] = inf16
            return carry

        lax.fori_loop(0, NCH, init_j, 0)
        idx_v[pl.ds(0, L)] = jnp.zeros((L,), jnp.int32)

        lane = lax.iota(jnp.int32, L)

        def step(t, k):
            lx = plsc.load_gather(xs_v, [k])
            ly = plsc.load_gather(ys_v, [k])
            lz = plsc.load_gather(zs_v, [k])

            best0 = jnp.full((L,), -jnp.inf, jnp.float32)
            bidx0 = jnp.full((L,), 2**30, jnp.int32)

            def chunk(s, carry):
                best, bidx = carry
                dx = xs_v[pl.ds(s, L)] - lx
                dy = ys_v[pl.ds(s, L)] - ly
                dz = zs_v[pl.ds(s, L)] - lz
                d = dx * dx + dy * dy + dz * dz
                nd = jnp.minimum(dist_v[pl.ds(s, L)], d)
                dist_v[pl.ds(s, L)] = nd
                idx = s + lane
                gt = nd > best
                eq = nd == best
                bidx = jnp.where(gt, idx,
                                 jnp.where(eq, jnp.minimum(bidx, idx), bidx))
                best = jnp.maximum(best, nd)
                return best, bidx

            best, bidx = plsc.parallel_loop(
                0, N, step=L, unroll=8, carry=(best0, bidx0))(chunk)

            maxv = jnp.max(best)
            cand = jnp.where(best == maxv, bidx, jnp.int32(2**30))
            knext = jnp.full((L,), jnp.min(cand), jnp.int32)
            plsc.store_scatter(idx_v, [jnp.full((L,), t, jnp.int32)],
                               knext, mask=lane == 0)
            return knext

        lax.fori_loop(1, M, step, jnp.zeros((L,), jnp.int32))

        def gout(j, carry):
            s = j * L
            iv = idx_v[pl.ds(s, L)]
            ox_v[pl.ds(s, L)] = plsc.load_gather(xs_v, [iv])
            oy_v[pl.ds(s, L)] = plsc.load_gather(ys_v, [iv])
            oz_v[pl.ds(s, L)] = plsc.load_gather(zs_v, [iv])
            return carry

        lax.fori_loop(0, M // L, gout, 0)

        pltpu.sync_copy(ox_v, y_hbm.at[row + 0])
        pltpu.sync_copy(oy_v, y_hbm.at[row + 1])
        pltpu.sync_copy(oz_v, y_hbm.at[row + 2])


@jax.jit
def _fps_sc(xr):
    mesh = plsc.VectorSubcoreMesh(core_axis_name="c", subcore_axis_name="s")
    f = functools.partial(
        pl.kernel,
        mesh=mesh,
        compiler_params=pltpu.CompilerParams(needs_layout_passes=False),
        out_type=jax.ShapeDtypeStruct((B * C, M), jnp.float32),
        scratch_types=[
            pltpu.VMEM((N,), jnp.float32),
            pltpu.VMEM((N,), jnp.float32),
            pltpu.VMEM((N,), jnp.float32),
            pltpu.VMEM((N,), jnp.float32),
            pltpu.VMEM((M,), jnp.int32),
            pltpu.VMEM((M,), jnp.float32),
            pltpu.VMEM((M,), jnp.float32),
            pltpu.VMEM((M,), jnp.float32),
        ],
    )(_fps_body)
    return f(xr)


def kernel(x):
    xr = x.reshape(B * C, N)
    yr = _fps_sc(xr)
    return yr.reshape(B, C, M)
